# speedup vs baseline: 2.7794x; 2.7794x over previous
import functools

import jax
import jax.numpy as jnp
from jax import lax
from jax.experimental import pallas as pl
from jax.experimental.pallas import tpu as pltpu

try:
    from jax.experimental.pallas import tpu_sc as plsc
    _HAS_SC = True
except ImportError:
    plsc = None
    _HAS_SC = False

EMB = 128
BATCH = 16384
NC = 2
NS = 16
NW = NC * NS
BPW = BATCH // NW
CHUNK = 128
NCHUNK = BPW // CHUNK

MLP_BLOCK = 2048


def _sc_gather(rid, pid, R_emb, P_emb):
    mesh = plsc.VectorSubcoreMesh(core_axis_name="c", subcore_axis_name="s")

    @functools.partial(
        pl.kernel,
        mesh=mesh,
        out_type=[
            jax.ShapeDtypeStruct((BATCH, EMB), jnp.float32),
            jax.ShapeDtypeStruct((BATCH, EMB), jnp.float32),
        ],
        scratch_types=[
            pltpu.VMEM((NCHUNK, CHUNK), jnp.int32),
            pltpu.VMEM((NCHUNK, CHUNK), jnp.int32),
            pltpu.VMEM((BPW, EMB), jnp.float32),
            pltpu.SemaphoreType.DMA,
        ],
    )
    def gather_k(rid_hbm, pid_hbm, R_hbm, P_hbm, rev_out, prod_out,
                 ridx_v, pidx_v, rows_v, sem):
        wid = lax.axis_index("s") * NC + lax.axis_index("c")
        base = wid * BPW
        pltpu.sync_copy(rid_hbm.at[wid], ridx_v)
        pltpu.sync_copy(pid_hbm.at[wid], pidx_v)
        copies = [
            pltpu.async_copy(
                R_hbm.at[ridx_v.at[j]],
                rows_v.at[pl.ds(j * CHUNK, CHUNK)], sem)
            for j in range(NCHUNK)
        ]
        for c in copies:
            c.wait()
        pltpu.sync_copy(rows_v, rev_out.at[pl.ds(base, BPW)])
        copies = [
            pltpu.async_copy(
                P_hbm.at[pidx_v.at[j]],
                rows_v.at[pl.ds(j * CHUNK, CHUNK)], sem)
            for j in range(NCHUNK)
        ]
        for c in copies:
            c.wait()
        pltpu.sync_copy(rows_v, prod_out.at[pl.ds(base, BPW)])

    return gather_k(rid, pid, R_emb, P_emb)


def _mlp_body(rev_ref, prod_ref, w1r_ref, w1p_ref, b1_ref, w2_ref, b2_ref,
              out_ref):
    h = jnp.dot(rev_ref[...], w1r_ref[...], preferred_element_type=jnp.float32)
    h = h + jnp.dot(prod_ref[...], w1p_ref[...],
                    preferred_element_type=jnp.float32)
    h = jnp.maximum(h + b1_ref[...], 0.0)
    out_ref[...] = jnp.sum(h * w2_ref[...], axis=1) + b2_ref[0, 0]


def _tc_mlp(rev, prod, W1, b1, W2, b2, *, interpret=False):
    w1r = W1[:EMB]
    w1p = W1[EMB:]
    b1r = b1.reshape(1, 64)
    w2r = W2.reshape(1, 64)
    b2r = b2.reshape(1, 1)
    grid = (BATCH // MLP_BLOCK,)
    return pl.pallas_call(
        _mlp_body,
        grid=grid,
        in_specs=[
            pl.BlockSpec((MLP_BLOCK, EMB), lambda i: (i, 0)),
            pl.BlockSpec((MLP_BLOCK, EMB), lambda i: (i, 0)),
            pl.BlockSpec((EMB, 64), lambda i: (0, 0)),
            pl.BlockSpec((EMB, 64), lambda i: (0, 0)),
            pl.BlockSpec((1, 64), lambda i: (0, 0)),
            pl.BlockSpec((1, 64), lambda i: (0, 0)),
            pl.BlockSpec(memory_space=pltpu.SMEM),
        ],
        out_specs=pl.BlockSpec((MLP_BLOCK,), lambda i: (i,)),
        out_shape=jax.ShapeDtypeStruct((BATCH,), jnp.float32),
        interpret=interpret,
    )(rev, prod, w1r, w1p, b1r, w2r, b2r)


def kernel(product_id, reviewer_id, R_emb, P_emb, W1, b1, W2, b2):
    rid = reviewer_id.astype(jnp.int32).reshape(NW, NCHUNK, CHUNK)
    pid = product_id.astype(jnp.int32).reshape(NW, NCHUNK, CHUNK)
    rev, prod = _sc_gather(rid, pid, R_emb, P_emb)
    return _tc_mlp(rev, prod, W1, b1, W2, b2)


# speedup vs baseline: 3.6403x; 1.3098x over previous
import functools

import jax
import jax.numpy as jnp
from jax import lax
from jax.experimental import pallas as pl
from jax.experimental.pallas import tpu as pltpu

try:
    from jax.experimental.pallas import tpu_sc as plsc
    _HAS_SC = True
except ImportError:
    plsc = None
    _HAS_SC = False

EMB = 128
BATCH = 16384
NC = 2
NS = 16
NW = NC * NS
BPW = BATCH // NW
CHUNK = 128
NCHUNK = BPW // CHUNK

MLP_BLOCK = 8192


def _sc_gather(rid, pid, R_emb, P_emb):
    mesh = plsc.VectorSubcoreMesh(core_axis_name="c", subcore_axis_name="s")

    @functools.partial(
        pl.kernel,
        mesh=mesh,
        out_type=[
            jax.ShapeDtypeStruct((BATCH, EMB), jnp.float32),
            jax.ShapeDtypeStruct((BATCH, EMB), jnp.float32),
        ],
        scratch_types=[
            pltpu.VMEM((NCHUNK, CHUNK), jnp.int32),
            pltpu.VMEM((NCHUNK, CHUNK), jnp.int32),
            pltpu.VMEM((BPW, EMB), jnp.float32),
            pltpu.SemaphoreType.DMA,
        ],
    )
    def gather_k(rid_hbm, pid_hbm, R_hbm, P_hbm, rev_out, prod_out,
                 ridx_v, pidx_v, rows_v, sem):
        wid = lax.axis_index("s") * NC + lax.axis_index("c")
        base = wid * BPW
        pltpu.sync_copy(rid_hbm.at[wid], ridx_v)
        pltpu.sync_copy(pid_hbm.at[wid], pidx_v)
        copies = [
            pltpu.async_copy(
                R_hbm.at[ridx_v.at[j]],
                rows_v.at[pl.ds(j * CHUNK, CHUNK)], sem)
            for j in range(NCHUNK)
        ]
        for c in copies:
            c.wait()
        pltpu.sync_copy(rows_v, rev_out.at[pl.ds(base, BPW)])
        copies = [
            pltpu.async_copy(
                P_hbm.at[pidx_v.at[j]],
                rows_v.at[pl.ds(j * CHUNK, CHUNK)], sem)
            for j in range(NCHUNK)
        ]
        for c in copies:
            c.wait()
        pltpu.sync_copy(rows_v, prod_out.at[pl.ds(base, BPW)])

    return gather_k(rid, pid, R_emb, P_emb)


def _mlp_body(rev_ref, prod_ref, w1r_ref, w1p_ref, b1_ref, w2_ref, b2_ref,
              out_ref):
    hT = lax.dot_general(w1r_ref[...], rev_ref[...],
                         (((0,), (1,)), ((), ())),
                         preferred_element_type=jnp.float32)
    hT = hT + lax.dot_general(w1p_ref[...], prod_ref[...],
                              (((0,), (1,)), ((), ())),
                              preferred_element_type=jnp.float32)
    hT = jnp.maximum(hT + b1_ref[...], 0.0)
    out_ref[...] = jnp.sum(hT * w2_ref[...], axis=0) + b2_ref[0, 0]


def _tc_mlp(rev, prod, W1, b1, W2, b2, *, interpret=False):
    w1r = W1[:EMB]
    w1p = W1[EMB:]
    b1c = b1.reshape(64, 1)
    w2c = W2
    b2r = b2.reshape(1, 1)
    grid = (BATCH // MLP_BLOCK,)
    return pl.pallas_call(
        _mlp_body,
        grid=grid,
        in_specs=[
            pl.BlockSpec((MLP_BLOCK, EMB), lambda i: (i, 0)),
            pl.BlockSpec((MLP_BLOCK, EMB), lambda i: (i, 0)),
            pl.BlockSpec((EMB, 64), lambda i: (0, 0)),
            pl.BlockSpec((EMB, 64), lambda i: (0, 0)),
            pl.BlockSpec((64, 1), lambda i: (0, 0)),
            pl.BlockSpec((64, 1), lambda i: (0, 0)),
            pl.BlockSpec(memory_space=pltpu.SMEM),
        ],
        out_specs=pl.BlockSpec((MLP_BLOCK,), lambda i: (i,)),
        out_shape=jax.ShapeDtypeStruct((BATCH,), jnp.float32),
        interpret=interpret,
    )(rev, prod, w1r, w1p, b1c, w2c, b2r)


def kernel(product_id, reviewer_id, R_emb, P_emb, W1, b1, W2, b2):
    rid = reviewer_id.astype(jnp.int32).reshape(NW, NCHUNK, CHUNK)
    pid = product_id.astype(jnp.int32).reshape(NW, NCHUNK, CHUNK)
    rev, prod = _sc_gather(rid, pid, R_emb, P_emb)
    return _tc_mlp(rev, prod, W1, b1, W2, b2)


# speedup vs baseline: 3.6828x; 1.0117x over previous
import functools

import jax
import jax.numpy as jnp
from jax import lax
from jax.experimental import pallas as pl
from jax.experimental.pallas import tpu as pltpu

try:
    from jax.experimental.pallas import tpu_sc as plsc
    _HAS_SC = True
except ImportError:
    plsc = None
    _HAS_SC = False

EMB = 128
BATCH = 16384
NC = 2
NS = 16
NW = NC * NS
BPW = BATCH // NW
CHUNK = 128
NCHUNK = BPW // CHUNK

MLP_BLOCK = 8192


NSTEP = 2 * NCHUNK
NSLOT = 7


def _sc_gather(rid, pid, R_emb, P_emb):
    mesh = plsc.VectorSubcoreMesh(core_axis_name="c", subcore_axis_name="s")

    @functools.partial(
        pl.kernel,
        mesh=mesh,
        out_type=[
            jax.ShapeDtypeStruct((BATCH, EMB), jnp.float32),
            jax.ShapeDtypeStruct((BATCH, EMB), jnp.float32),
        ],
        scratch_types=[
            pltpu.VMEM((BPW,), jnp.int32),
            pltpu.VMEM((BPW,), jnp.int32),
            pltpu.VMEM((NSLOT * CHUNK, EMB), jnp.float32),
            pltpu.SemaphoreType.DMA,
            pltpu.SemaphoreType.DMA,
        ],
    )
    def gather_k(rid_hbm, pid_hbm, R_hbm, P_hbm, rev_out, prod_out,
                 ridx_v, pidx_v, rows_v, gsem, osem):
        wid = lax.axis_index("s") * NC + lax.axis_index("c")
        base = wid * BPW

        i1 = pltpu.async_copy(rid_hbm.at[pl.ds(base, BPW)], ridx_v, gsem)
        i2 = pltpu.async_copy(pid_hbm.at[pl.ds(base, BPW)], pidx_v, gsem)
        i1.wait()
        i2.wait()

        def fire(k):
            slot = rows_v.at[pl.ds((k % NSLOT) * CHUNK, CHUNK)]
            if k < NCHUNK:
                idx = ridx_v.at[pl.ds(k * CHUNK, CHUNK)]
                return pltpu.async_copy(R_hbm.at[idx], slot, gsem)
            idx = pidx_v.at[pl.ds((k - NCHUNK) * CHUNK, CHUNK)]
            return pltpu.async_copy(P_hbm.at[idx], slot, gsem)

        def fire_out(k):
            slot = rows_v.at[pl.ds((k % NSLOT) * CHUNK, CHUNK)]
            if k < NCHUNK:
                dst = rev_out.at[pl.ds(base + k * CHUNK, CHUNK)]
            else:
                dst = prod_out.at[pl.ds(base + (k - NCHUNK) * CHUNK, CHUNK)]
            return pltpu.async_copy(slot, dst, osem)

        gathers = [fire(k) for k in range(NSLOT)]
        outs = []
        for k in range(NSTEP):
            if k >= NSLOT:
                outs[k - NSLOT].wait()
                gathers.append(fire(k))
            gathers[k].wait()
            outs.append(fire_out(k))
        for k in range(NSTEP - NSLOT, NSTEP):
            outs[k].wait()

    return gather_k(rid, pid, R_emb, P_emb)


def _mlp_body(rev_ref, prod_ref, w1r_ref, w1p_ref, b1_ref, w2_ref, b2_ref,
              out_ref):
    hT = lax.dot_general(w1r_ref[...], rev_ref[...],
                         (((0,), (1,)), ((), ())),
                         preferred_element_type=jnp.float32)
    hT = hT + lax.dot_general(w1p_ref[...], prod_ref[...],
                              (((0,), (1,)), ((), ())),
                              preferred_element_type=jnp.float32)
    hT = jnp.maximum(hT + b1_ref[...], 0.0)
    out_ref[...] = jnp.sum(hT * w2_ref[...], axis=0) + b2_ref[0, 0]


def _tc_mlp(rev, prod, W1, b1, W2, b2, *, interpret=False):
    w1r = W1[:EMB]
    w1p = W1[EMB:]
    b1c = b1.reshape(64, 1)
    w2c = W2
    b2r = b2.reshape(1, 1)
    grid = (BATCH // MLP_BLOCK,)
    return pl.pallas_call(
        _mlp_body,
        grid=grid,
        in_specs=[
            pl.BlockSpec((MLP_BLOCK, EMB), lambda i: (i, 0)),
            pl.BlockSpec((MLP_BLOCK, EMB), lambda i: (i, 0)),
            pl.BlockSpec((EMB, 64), lambda i: (0, 0)),
            pl.BlockSpec((EMB, 64), lambda i: (0, 0)),
            pl.BlockSpec((64, 1), lambda i: (0, 0)),
            pl.BlockSpec((64, 1), lambda i: (0, 0)),
            pl.BlockSpec(memory_space=pltpu.SMEM),
        ],
        out_specs=pl.BlockSpec((MLP_BLOCK,), lambda i: (i,)),
        out_shape=jax.ShapeDtypeStruct((BATCH,), jnp.float32),
        interpret=interpret,
    )(rev, prod, w1r, w1p, b1c, w2c, b2r)


def kernel(product_id, reviewer_id, R_emb, P_emb, W1, b1, W2, b2):
    rid = reviewer_id.astype(jnp.int32)
    pid = product_id.astype(jnp.int32)
    rev, prod = _sc_gather(rid, pid, R_emb, P_emb)
    return _tc_mlp(rev, prod, W1, b1, W2, b2)
